# Initial kernel scaffold; baseline (speedup 1.0000x reference)
import jax
import jax.numpy as jnp
from jax.experimental import pallas as pl


def kernel(x, edge_index, batch, target_embedding, W1, a_src1, a_dst1, b1, W2, a_src2, a_dst2, b2, Ww, bw, Wxt, bxt, Wfc1, bfc1, Wfc2, bfc2, Wout, bout):
    raise NotImplementedError("write your pallas kernel here")


# speedup vs baseline: 11.3438x; 11.3438x over previous
import functools
import jax
import jax.numpy as jnp
from jax import lax
from jax.experimental import pallas as pl
from jax.experimental.pallas import tpu as pltpu
from jax.experimental.pallas import tpu_sc as plsc

N = 10000
NPAD = 10112
E = 160000
H = 10
C1 = 78
CP1 = 80
C2 = 128
CP2 = 128
G = 64
NBLK = 1264
NB = 8
EWP1 = 10112
NCH1 = 79
EW2 = 5120
NCH2 = 40
EPAD = 163840
SENT = 10000
STRIPE = NPAD // 16


def _lrelu(x):
    return jnp.where(x > 0, x, 0.2 * x)


def _tc1a_body(x_ref, w1_ref, asr_ref, adr_ref, hpad_ref, as1_ref, ad1_ref):
    x = x_ref[...]
    w1 = w1_ref[...]
    ones = jnp.ones((NBLK, 1), jnp.float32)
    zeros = jnp.zeros((NBLK, 1), jnp.float32)
    as_rows = []
    ad_rows = []
    for hd in range(H):
        w = w1[:, hd * C1:(hd + 1) * C1]
        hh = jnp.dot(x, w, preferred_element_type=jnp.float32, precision=lax.Precision.HIGHEST)
        hpad_ref[hd] = jnp.concatenate([hh, ones, zeros], axis=1)
        as_rows.append(jnp.sum(hh * asr_ref[hd, :][None, :], axis=1))
        ad_rows.append(jnp.sum(hh * adr_ref[hd, :][None, :], axis=1))
    as1_ref[0] = jnp.stack(as_rows, axis=0)
    ad1_ref[0] = jnp.stack(ad_rows, axis=0)


def _tc1a(xpad, W1, a_src1, a_dst1):
    return pl.pallas_call(
        _tc1a_body,
        grid=(NPAD // NBLK,),
        in_specs=[
            pl.BlockSpec((NBLK, C1), lambda i: (i, 0)),
            pl.BlockSpec((C1, H * C1), lambda i: (0, 0)),
            pl.BlockSpec((H, C1), lambda i: (0, 0)),
            pl.BlockSpec((H, C1), lambda i: (0, 0)),
        ],
        out_specs=[
            pl.BlockSpec((H, NBLK, CP1), lambda i: (0, i, 0)),
            pl.BlockSpec((1, H, NBLK), lambda i: (i, 0, 0)),
            pl.BlockSpec((1, H, NBLK), lambda i: (i, 0, 0)),
        ],
        out_shape=[
            jax.ShapeDtypeStruct((H, NPAD, CP1), jnp.float32),
            jax.ShapeDtypeStruct((NB, H, NBLK), jnp.float32),
            jax.ShapeDtypeStruct((NB, H, NBLK), jnp.float32),
        ],
    )(xpad, W1, a_src1, a_dst1)


def _tc1b_body(asf_ref, asb_ref, adb_ref, hpad_ref, b1_ref, acc0_ref):
    A = jnp.max(asf_ref[...], axis=(0, 2))
    asb = asb_ref[0]
    adb = adb_ref[0]
    B = _lrelu(adb + A[:, None])
    exs = jnp.exp(_lrelu(asb + adb) - B)
    for hd in range(H):
        acc0_ref[hd] = hpad_ref[hd] * exs[hd, :][:, None]
    b1_ref[0] = B


def _tc1b(as1, ad1, hpad):
    return pl.pallas_call(
        _tc1b_body,
        grid=(NPAD // NBLK,),
        in_specs=[
            pl.BlockSpec((NB, H, NBLK), lambda i: (0, 0, 0)),
            pl.BlockSpec((1, H, NBLK), lambda i: (i, 0, 0)),
            pl.BlockSpec((1, H, NBLK), lambda i: (i, 0, 0)),
            pl.BlockSpec((H, NBLK, CP1), lambda i: (0, i, 0)),
        ],
        out_specs=[
            pl.BlockSpec((1, H, NBLK), lambda i: (i, 0, 0)),
            pl.BlockSpec((H, NBLK, CP1), lambda i: (0, i, 0)),
        ],
        out_shape=[
            jax.ShapeDtypeStruct((NB, H, NBLK), jnp.float32),
            jax.ShapeDtypeStruct((H, NPAD, CP1), jnp.float32),
        ],
    )(as1, as1, ad1, hpad)


def _sc1_body(srcp, dstp, asf, adf, bf, hpadf, acc0f, out,
              asv, adv, bv, srcv, dstv, gidx, exv, rows, sacc, sem):
    c = lax.axis_index("c")
    s = lax.axis_index("s")
    zero16 = jnp.zeros((16,), jnp.int32)
    for hi in range(H // 2):
        hd = c * (H // 2) + hi
        base = hd * NPAD
        pltpu.sync_copy(asf.at[pl.ds(base, NPAD)], asv)
        pltpu.sync_copy(adf.at[pl.ds(base, NPAD)], adv)
        pltpu.sync_copy(bf.at[pl.ds(base, NPAD)], bv)
        pltpu.sync_copy(acc0f.at[pl.ds(base + s * STRIPE, STRIPE)],
                        sacc.at[pl.ds(s * STRIPE, STRIPE)])
        plsc.subcore_barrier()

        def chunk_body(ch, _):
            eb = s * EWP1 + ch * 128
            pltpu.sync_copy(srcp.at[pl.ds(eb, 128)], srcv)
            pltpu.sync_copy(dstp.at[pl.ds(eb, 128)], dstv)
            for v in range(8):
                sl = pl.ds(v * 16, 16)
                si = srcv[sl]
                di = dstv[sl]
                a_s = plsc.load_gather(asv, [si])
                a_d = plsc.load_gather(adv, [di])
                b_d = plsc.load_gather(bv, [di])
                e = _lrelu(a_s + a_d)
                exv[sl] = jnp.exp(e - b_d)
                gidx[sl] = si + base
            pltpu.async_copy(hpadf.at[gidx], rows, sem).wait()

            def scale_body(i2, _):
                spl = plsc.load_gather(exv, [zero16 + i2])
                for t in range(CP1 // 16):
                    sl2 = pl.ds(t * 16, 16)
                    rows[i2, sl2] = rows[i2, sl2] * spl
                return 0
            lax.fori_loop(0, 128, scale_body, 0)
            pltpu.sync_copy(rows, sacc.at[dstv], add=True)
            return 0
        lax.fori_loop(0, NCH1, chunk_body, 0)
        plsc.subcore_barrier()
        pltpu.sync_copy(sacc.at[pl.ds(s * STRIPE, STRIPE)],
                        out.at[pl.ds(base + s * STRIPE, STRIPE)])
        plsc.subcore_barrier()


def _sc1(srcp, dstp, asf, adf, bf, hpadf, acc0f):
    mesh = plsc.VectorSubcoreMesh(core_axis_name="c", subcore_axis_name="s")
    fn = functools.partial(
        pl.kernel, mesh=mesh,
        out_type=jax.ShapeDtypeStruct((H * NPAD, CP1), jnp.float32),
        compiler_params=pltpu.CompilerParams(needs_layout_passes=False,
                                             use_tc_tiling_on_sc=False),
        scratch_types=[
            pltpu.VMEM((NPAD,), jnp.float32),
            pltpu.VMEM((NPAD,), jnp.float32),
            pltpu.VMEM((NPAD,), jnp.float32),
            pltpu.VMEM((128,), jnp.int32),
            pltpu.VMEM((128,), jnp.int32),
            pltpu.VMEM((128,), jnp.int32),
            pltpu.VMEM((128,), jnp.float32),
            pltpu.VMEM((128, CP1), jnp.float32),
            pltpu.VMEM_SHARED((NPAD, CP1), jnp.float32),
            pltpu.SemaphoreType.DMA,
        ])(_sc1_body)
    return fn(srcp, dstp, asf, adf, bf, hpadf, acc0f)


def _tc2a_body(acc_ref, b1r_ref, w2_ref, as2r_ref, ad2r_ref,
               hpad2_ref, asad_ref):
    h2 = jnp.zeros((NBLK, C2), jnp.float32)
    w2 = w2_ref[...]
    for hd in range(H):
        acc = acc_ref[hd]
        hn = acc[:, 0:C1] / (acc[:, C1:C1 + 1] + 1e-30) + b1r_ref[hd, :][None, :]
        helu = jnp.where(hn > 0, hn, jnp.exp(hn) - 1.0)
        h2 = h2 + jnp.dot(helu, w2[hd * C1:(hd + 1) * C1, :],
                          preferred_element_type=jnp.float32, precision=lax.Precision.HIGHEST)
    as2 = jnp.sum(h2 * as2r_ref[0, :][None, :], axis=1, keepdims=True)
    ad2 = jnp.sum(h2 * ad2r_ref[0, :][None, :], axis=1, keepdims=True)
    hpad2_ref[...] = h2
    asad_ref[...] = jnp.concatenate([as2, ad2], axis=1)


def _tc2a(acc1, b1r, W2, a_src2, a_dst2):
    return pl.pallas_call(
        _tc2a_body,
        grid=(NPAD // NBLK,),
        in_specs=[
            pl.BlockSpec((H, NBLK, CP1), lambda i: (0, i, 0)),
            pl.BlockSpec((H, C1), lambda i: (0, 0)),
            pl.BlockSpec((H * C1, C2), lambda i: (0, 0)),
            pl.BlockSpec((1, C2), lambda i: (0, 0)),
            pl.BlockSpec((1, C2), lambda i: (0, 0)),
        ],
        out_specs=[
            pl.BlockSpec((NBLK, CP2), lambda i: (i, 0)),
            pl.BlockSpec((NBLK, 2), lambda i: (i, 0)),
        ],
        out_shape=[
            jax.ShapeDtypeStruct((NPAD, CP2), jnp.float32),
            jax.ShapeDtypeStruct((NPAD, 2), jnp.float32),
        ],
    )(acc1, b1r, W2, a_src2, a_dst2)


def _tc2b_body(asadf_ref, asadb_ref, hpad2_ref, b2c_ref, acc20_ref, den0_ref):
    A = jnp.max(asadf_ref[:, 0])
    asb = asadb_ref[:, 0:1]
    adb = asadb_ref[:, 1:2]
    B = _lrelu(adb + A)
    exs = jnp.exp(_lrelu(asb + adb) - B)
    acc20_ref[...] = hpad2_ref[...] * exs
    den0_ref[...] = exs
    b2c_ref[...] = B


def _tc2b(asad, hpad2):
    return pl.pallas_call(
        _tc2b_body,
        grid=(NPAD // NBLK,),
        in_specs=[
            pl.BlockSpec((NPAD, 2), lambda i: (0, 0)),
            pl.BlockSpec((NBLK, 2), lambda i: (i, 0)),
            pl.BlockSpec((NBLK, CP2), lambda i: (i, 0)),
        ],
        out_specs=[
            pl.BlockSpec((NBLK, 1), lambda i: (i, 0)),
            pl.BlockSpec((NBLK, CP2), lambda i: (i, 0)),
            pl.BlockSpec((NBLK, 1), lambda i: (i, 0)),
        ],
        out_shape=[
            jax.ShapeDtypeStruct((NPAD, 1), jnp.float32),
            jax.ShapeDtypeStruct((NPAD, CP2), jnp.float32),
            jax.ShapeDtypeStruct((NPAD, 1), jnp.float32),
        ],
    )(asad, asad, hpad2)


def _sc2_body(srcp, dstp, asf, adf, bf, hpadf, acc0f, den0f, out, dout,
              asv, adv, bv, srcv, dstv, exv, rows, sacc, sden, sem):
    c = lax.axis_index("c")
    s = lax.axis_index("s")
    g = c * 16 + s
    zero16 = jnp.zeros((16,), jnp.int32)
    pltpu.sync_copy(asf.at[pl.ds(0, NPAD)], asv)
    pltpu.sync_copy(adf.at[pl.ds(0, NPAD)], adv)
    pltpu.sync_copy(bf.at[pl.ds(0, NPAD)], bv)
    pltpu.sync_copy(acc0f.at[c, pl.ds(s * STRIPE, STRIPE)],
                    sacc.at[pl.ds(s * STRIPE, STRIPE)])
    pltpu.sync_copy(den0f.at[c, pl.ds(s * STRIPE, STRIPE)],
                    sden.at[pl.ds(s * STRIPE, STRIPE)])
    plsc.subcore_barrier()

    def chunk_body(ch, _):
        eb = g * EW2 + ch * 128
        pltpu.sync_copy(srcp.at[pl.ds(eb, 128)], srcv)
        pltpu.sync_copy(dstp.at[pl.ds(eb, 128)], dstv)
        for v in range(8):
            sl = pl.ds(v * 16, 16)
            si = srcv[sl]
            di = dstv[sl]
            a_s = plsc.load_gather(asv, [si])
            a_d = plsc.load_gather(adv, [di])
            b_d = plsc.load_gather(bv, [di])
            e = _lrelu(a_s + a_d)
            exv[sl] = jnp.exp(e - b_d)
        pltpu.async_copy(hpadf.at[srcv], rows, sem).wait()

        def scale_body(i2, _):
            spl = plsc.load_gather(exv, [zero16 + i2])
            for t in range(CP2 // 16):
                sl2 = pl.ds(t * 16, 16)
                rows[i2, sl2] = rows[i2, sl2] * spl
            return 0
        lax.fori_loop(0, 128, scale_body, 0)
        pltpu.sync_copy(rows, sacc.at[dstv], add=True)
        pltpu.sync_copy(exv, sden.at[dstv], add=True)
        return 0
    lax.fori_loop(0, NCH2, chunk_body, 0)
    plsc.subcore_barrier()
    pltpu.sync_copy(sacc.at[pl.ds(s * STRIPE, STRIPE)],
                    out.at[c, pl.ds(s * STRIPE, STRIPE)])
    pltpu.sync_copy(sden.at[pl.ds(s * STRIPE, STRIPE)],
                    dout.at[c, pl.ds(s * STRIPE, STRIPE)])


def _sc2(srcp, dstp, as2f, ad2f, b2f, hpad2, acc20full, den0full):
    mesh = plsc.VectorSubcoreMesh(core_axis_name="c", subcore_axis_name="s")
    fn = functools.partial(
        pl.kernel, mesh=mesh,
        out_type=[jax.ShapeDtypeStruct((2, NPAD, CP2), jnp.float32),
                  jax.ShapeDtypeStruct((2, NPAD), jnp.float32)],
        compiler_params=pltpu.CompilerParams(needs_layout_passes=False,
                                             use_tc_tiling_on_sc=False),
        scratch_types=[
            pltpu.VMEM((NPAD,), jnp.float32),
            pltpu.VMEM((NPAD,), jnp.float32),
            pltpu.VMEM((NPAD,), jnp.float32),
            pltpu.VMEM((128,), jnp.int32),
            pltpu.VMEM((128,), jnp.int32),
            pltpu.VMEM((128,), jnp.float32),
            pltpu.VMEM((128, CP2), jnp.float32),
            pltpu.VMEM_SHARED((NPAD, CP2), jnp.float32),
            pltpu.VMEM_SHARED((NPAD,), jnp.float32),
            pltpu.SemaphoreType.DMA,
        ])(_sc2_body)
    return fn(srcp, dstp, as2f, ad2f, b2f, hpad2, acc20full, den0full)


def _tc3a_body(a0_ref, a1_ref, d_ref, b2r_ref, ww_ref, bw_ref,
               h_ref, hw_ref):
    acc = a0_ref[0] + a1_ref[0]
    den = (d_ref[0, 0, :] + d_ref[0, 1, :])[:, None]
    h = acc / (den + 1e-30) + b2r_ref[0, :][None, :]
    h = jnp.maximum(h, 0.0)
    logit = jnp.dot(h, ww_ref[...], preferred_element_type=jnp.float32, precision=lax.Precision.HIGHEST) + bw_ref[0, 0]
    w = 1.0 / (1.0 + jnp.exp(-logit))
    h_ref[...] = h
    hw_ref[...] = h * w


def _tc3a(accL2, denL2, b2r, Ww, bw):
    return pl.pallas_call(
        _tc3a_body,
        grid=(NPAD // NBLK,),
        in_specs=[
            pl.BlockSpec((1, NBLK, CP2), lambda i: (0, i, 0)),
            pl.BlockSpec((1, NBLK, CP2), lambda i: (1, i, 0)),
            pl.BlockSpec((1, 2, NBLK), lambda i: (i, 0, 0)),
            pl.BlockSpec((1, C2), lambda i: (0, 0)),
            pl.BlockSpec((C2, 1), lambda i: (0, 0)),
            pl.BlockSpec((1, 1), lambda i: (0, 0)),
        ],
        out_specs=[
            pl.BlockSpec((NBLK, C2), lambda i: (i, 0)),
            pl.BlockSpec((NBLK, C2), lambda i: (i, 0)),
        ],
        out_shape=[
            jax.ShapeDtypeStruct((NPAD, C2), jnp.float32),
            jax.ShapeDtypeStruct((NPAD, C2), jnp.float32),
        ],
    )(accL2, accL2, denL2.reshape(2, NB, NBLK).transpose(1, 0, 2), b2r, Ww, bw)


def _tc3b_body(h_ref, hw_ref, batch_ref, batchc_ref, hsum_ref, hmax8_ref):
    bt = batch_ref[...]
    gi = lax.broadcasted_iota(jnp.int32, (G, NPAD), 0)
    oh = (bt == gi)
    hsum_ref[...] = jnp.dot(oh.astype(jnp.float32), hw_ref[...],
                            preferred_element_type=jnp.float32, precision=lax.Precision.HIGHEST)
    h = h_ref[...]
    btc = batchc_ref[...]
    ninf = jnp.float32(-jnp.inf)

    def body(g, _):
        mask = (btc == g)
        row = jnp.max(jnp.where(mask, h, ninf), axis=0, keepdims=True)
        row = jnp.where(jnp.isfinite(row), row, 0.0)
        hmax8_ref[pl.ds(pl.multiple_of(g * 8, 8), 8), :] = (
            jnp.broadcast_to(row, (8, C2)))
        return 0
    lax.fori_loop(0, G, body, 0)


def _tc3b(h, hw, batchr):
    return pl.pallas_call(
        _tc3b_body,
        grid=(1,),
        in_specs=[
            pl.BlockSpec((NPAD, C2), lambda i: (0, 0)),
            pl.BlockSpec((NPAD, C2), lambda i: (0, 0)),
            pl.BlockSpec((1, NPAD), lambda i: (0, 0)),
            pl.BlockSpec((NPAD, 1), lambda i: (0, 0)),
        ],
        out_specs=[
            pl.BlockSpec((G, C2), lambda i: (0, 0)),
            pl.BlockSpec((G * 8, C2), lambda i: (0, 0)),
        ],
        out_shape=[
            jax.ShapeDtypeStruct((G, C2), jnp.float32),
            jax.ShapeDtypeStruct((G * 8, C2), jnp.float32),
        ],
    )(h, hw, batchr, batchr.reshape(NPAD, 1))


def _tc3c_body(hg_ref, te_ref, wxt_ref, bxt_ref, w1_ref, bf1_ref,
               w2_ref, bf2_ref, wo_ref, bo_ref, out_ref):
    xt = jnp.dot(te_ref[...], wxt_ref[...],
                 preferred_element_type=jnp.float32, precision=lax.Precision.HIGHEST) + bxt_ref[0, :][None, :]
    xc = jnp.concatenate([hg_ref[...], xt], axis=1)
    y = jnp.maximum(jnp.dot(xc, w1_ref[...], preferred_element_type=jnp.float32, precision=lax.Precision.HIGHEST)
                    + bf1_ref[0, :][None, :], 0.0)
    y = jnp.maximum(jnp.dot(y, w2_ref[...], preferred_element_type=jnp.float32, precision=lax.Precision.HIGHEST)
                    + bf2_ref[0, :][None, :], 0.0)
    out_ref[...] = jnp.dot(y, wo_ref[...],
                           preferred_element_type=jnp.float32, precision=lax.Precision.HIGHEST) + bo_ref[0, 0]


def _tc3c(hg, te, Wxt, bxt, Wfc1, bfc1, Wfc2, bfc2, Wout, bout):
    specs = [
        pl.BlockSpec((G, 2 * C2), lambda i: (0, 0)),
        pl.BlockSpec((G, 1280), lambda i: (0, 0)),
        pl.BlockSpec((1280, 256), lambda i: (0, 0)),
        pl.BlockSpec((1, 256), lambda i: (0, 0)),
        pl.BlockSpec((512, 1024), lambda i: (0, 0)),
        pl.BlockSpec((1, 1024), lambda i: (0, 0)),
        pl.BlockSpec((1024, 256), lambda i: (0, 0)),
        pl.BlockSpec((1, 256), lambda i: (0, 0)),
        pl.BlockSpec((256, 1), lambda i: (0, 0)),
        pl.BlockSpec((1, 1), lambda i: (0, 0)),
    ]
    return pl.pallas_call(
        _tc3c_body,
        grid=(1,),
        in_specs=specs,
        out_specs=pl.BlockSpec((G, 1), lambda i: (0, 0)),
        out_shape=jax.ShapeDtypeStruct((G, 1), jnp.float32),
    )(hg, te, Wxt, bxt.reshape(1, 256), Wfc1, bfc1.reshape(1, 1024),
      Wfc2, bfc2.reshape(1, 256), Wout, bout.reshape(1, 1))


def kernel(x, edge_index, batch, target_embedding, W1, a_src1, a_dst1, b1,
           W2, a_src2, a_dst2, b2, Ww, bw, Wxt, bxt, Wfc1, bfc1, Wfc2, bfc2,
           Wout, bout):
    xpad = jnp.zeros((NPAD, C1), jnp.float32).at[:N].set(x)
    ei = edge_index.astype(jnp.int32)
    srcp = jnp.full((EPAD,), SENT, jnp.int32).at[:E].set(ei[0])
    dstp = jnp.full((EPAD,), SENT, jnp.int32).at[:E].set(ei[1])

    hpad, as1, ad1 = _tc1a(xpad, W1, a_src1, a_dst1)
    B1, acc0 = _tc1b(as1, ad1, hpad)
    asf = as1.transpose(1, 0, 2).reshape(-1)
    adf = ad1.transpose(1, 0, 2).reshape(-1)
    bf = B1.transpose(1, 0, 2).reshape(-1)
    acc1 = _sc1(srcp, dstp, asf, adf, bf,
                hpad.reshape(H * NPAD, CP1), acc0.reshape(H * NPAD, CP1))
    acc1 = acc1.reshape(H, NPAD, CP1)

    hpad2, asad2 = _tc2a(acc1, b1.reshape(H, C1), W2, a_src2, a_dst2)
    B2, acc20, den0 = _tc2b(asad2, hpad2)
    acc20full = jnp.stack([acc20, jnp.zeros_like(acc20)], axis=0)
    den0full = jnp.stack([den0.reshape(-1), jnp.zeros((NPAD,), jnp.float32)],
                         axis=0)
    accL2, denL2 = _sc2(srcp, dstp, asad2[:, 0], asad2[:, 1], B2.reshape(-1),
                        hpad2, acc20full, den0full)

    batchr = jnp.full((1, NPAD), jnp.int32(1 << 30)).at[0, :N].set(
        batch.astype(jnp.int32))
    h, hw = _tc3a(accL2, denL2, b2.reshape(1, C2), Ww, bw.reshape(1, 1))
    hsum, hmax8 = _tc3b(h, hw, batchr)
    hg = jnp.concatenate([hsum, hmax8.reshape(G, 8, C2)[:, 0, :]], axis=1)
    return _tc3c(hg, target_embedding, Wxt, bxt, Wfc1, bfc1, Wfc2, bfc2,
                 Wout, bout)


# speedup vs baseline: 14.2712x; 1.2581x over previous
import functools
import jax
import jax.numpy as jnp
from jax import lax
from jax.experimental import pallas as pl
from jax.experimental.pallas import tpu as pltpu
from jax.experimental.pallas import tpu_sc as plsc

N = 10000
NPAD = 10112
E = 160000
H = 10
C1 = 78
CP1 = 80
C2 = 128
CP2 = 128
G = 64
NBLK = 1264
NB = 8
EWP1 = 10112
NCH1 = 79
EW2 = 5120
NCH2 = 40
EPAD = 163840
SENT = 10000
STRIPE = NPAD // 16


def _lrelu(x):
    return jnp.where(x > 0, x, 0.2 * x)


def _tc1a_body(x_ref, w1_ref, asr_ref, adr_ref, hpad_ref, as1_ref, ad1_ref):
    x = x_ref[...]
    w1 = w1_ref[...]
    ones = jnp.ones((NBLK, 1), jnp.float32)
    zeros = jnp.zeros((NBLK, 1), jnp.float32)
    as_rows = []
    ad_rows = []
    for hd in range(H):
        w = w1[:, hd * C1:(hd + 1) * C1]
        hh = jnp.dot(x, w, preferred_element_type=jnp.float32, precision=lax.Precision.HIGHEST)
        hpad_ref[hd] = jnp.concatenate([hh, ones, zeros], axis=1)
        as_rows.append(jnp.sum(hh * asr_ref[hd, :][None, :], axis=1))
        ad_rows.append(jnp.sum(hh * adr_ref[hd, :][None, :], axis=1))
    as1_ref[0] = jnp.stack(as_rows, axis=0)
    ad1_ref[0] = jnp.stack(ad_rows, axis=0)


def _tc1a(xpad, W1, a_src1, a_dst1):
    return pl.pallas_call(
        _tc1a_body,
        grid=(NPAD // NBLK,),
        in_specs=[
            pl.BlockSpec((NBLK, C1), lambda i: (i, 0)),
            pl.BlockSpec((C1, H * C1), lambda i: (0, 0)),
            pl.BlockSpec((H, C1), lambda i: (0, 0)),
            pl.BlockSpec((H, C1), lambda i: (0, 0)),
        ],
        out_specs=[
            pl.BlockSpec((H, NBLK, CP1), lambda i: (0, i, 0)),
            pl.BlockSpec((1, H, NBLK), lambda i: (i, 0, 0)),
            pl.BlockSpec((1, H, NBLK), lambda i: (i, 0, 0)),
        ],
        out_shape=[
            jax.ShapeDtypeStruct((H, NPAD, CP1), jnp.float32),
            jax.ShapeDtypeStruct((NB, H, NBLK), jnp.float32),
            jax.ShapeDtypeStruct((NB, H, NBLK), jnp.float32),
        ],
    )(xpad, W1, a_src1, a_dst1)


def _tc1b_body(asf_ref, asb_ref, adb_ref, hpad_ref, b1_ref, acc0_ref):
    A = jnp.max(asf_ref[...], axis=(0, 2))
    asb = asb_ref[0]
    adb = adb_ref[0]
    B = _lrelu(adb + A[:, None])
    exs = jnp.exp(_lrelu(asb + adb) - B)
    for hd in range(H):
        acc0_ref[hd] = hpad_ref[hd] * exs[hd, :][:, None]
    b1_ref[0] = B


def _tc1b(as1, ad1, hpad):
    return pl.pallas_call(
        _tc1b_body,
        grid=(NPAD // NBLK,),
        in_specs=[
            pl.BlockSpec((NB, H, NBLK), lambda i: (0, 0, 0)),
            pl.BlockSpec((1, H, NBLK), lambda i: (i, 0, 0)),
            pl.BlockSpec((1, H, NBLK), lambda i: (i, 0, 0)),
            pl.BlockSpec((H, NBLK, CP1), lambda i: (0, i, 0)),
        ],
        out_specs=[
            pl.BlockSpec((1, H, NBLK), lambda i: (i, 0, 0)),
            pl.BlockSpec((H, NBLK, CP1), lambda i: (0, i, 0)),
        ],
        out_shape=[
            jax.ShapeDtypeStruct((NB, H, NBLK), jnp.float32),
            jax.ShapeDtypeStruct((H, NPAD, CP1), jnp.float32),
        ],
    )(as1, as1, ad1, hpad)


def _sc1_body(srcp, dstp, asf, adf, bf, hpadf, acc0f, out,
              asv, adv, bv,
              srcv0, dstv0, gidx0, exv0, rows0,
              srcv1, dstv1, gidx1, exv1, rows1,
              sacc, semg0, semg1):
    c = lax.axis_index("c")
    s = lax.axis_index("s")
    zero16 = jnp.zeros((16,), jnp.int32)
    bufs = ((srcv0, dstv0, gidx0, exv0, rows0, semg0),
            (srcv1, dstv1, gidx1, exv1, rows1, semg1))
    for hi in range(H // 2):
        hd = c * (H // 2) + hi
        base = hd * NPAD
        pltpu.sync_copy(asf.at[pl.ds(base, NPAD)], asv)
        pltpu.sync_copy(adf.at[pl.ds(base, NPAD)], adv)
        pltpu.sync_copy(bf.at[pl.ds(base, NPAD)], bv)
        pltpu.sync_copy(acc0f.at[pl.ds(base + s * STRIPE, STRIPE)],
                        sacc.at[pl.ds(s * STRIPE, STRIPE)])
        plsc.subcore_barrier()

        def stage(ch, b, base=base):
            srcv, dstv, gidx, exv, rows, semg = bufs[b]

            @pl.when(ch < NCH1)
            def _():
                eb = s * EWP1 + ch * 128
                pltpu.sync_copy(srcp.at[pl.ds(eb, 128)], srcv)
                pltpu.sync_copy(dstp.at[pl.ds(eb, 128)], dstv)
                for v in range(8):
                    sl = pl.ds(v * 16, 16)
                    si = srcv[sl]
                    di = dstv[sl]
                    a_s = plsc.load_gather(asv, [si])
                    a_d = plsc.load_gather(adv, [di])
                    b_d = plsc.load_gather(bv, [di])
                    e = _lrelu(a_s + a_d)
                    exv[sl] = jnp.exp(e - b_d)
                    gidx[sl] = si + base
                pltpu.async_copy(hpadf.at[gidx], rows, semg)

        def process(ch, b):
            srcv, dstv, gidx, exv, rows, semg = bufs[b]

            @pl.when(ch < NCH1)
            def _():
                pltpu.make_async_copy(hpadf.at[gidx], rows, semg).wait()

                def scale_body(i2, _):
                    spl = plsc.load_gather(exv, [zero16 + i2])
                    for t in range(CP1 // 16):
                        sl2 = pl.ds(t * 16, 16)
                        rows[i2, sl2] = rows[i2, sl2] * spl
                    return 0
                lax.fori_loop(0, 128, scale_body, 0)
                pltpu.sync_copy(rows, sacc.at[dstv], add=True)

        stage(0, 0)

        def super_body(ss, _):
            ch0 = ss * 2
            stage(ch0 + 1, 1)
            process(ch0, 0)
            stage(ch0 + 2, 0)
            process(ch0 + 1, 1)
            return 0
        lax.fori_loop(0, (NCH1 + 1) // 2, super_body, 0)
        plsc.subcore_barrier()
        pltpu.sync_copy(sacc.at[pl.ds(s * STRIPE, STRIPE)],
                        out.at[pl.ds(base + s * STRIPE, STRIPE)])
        plsc.subcore_barrier()


def _sc1(srcp, dstp, asf, adf, bf, hpadf, acc0f):
    mesh = plsc.VectorSubcoreMesh(core_axis_name="c", subcore_axis_name="s")
    fn = functools.partial(
        pl.kernel, mesh=mesh,
        out_type=jax.ShapeDtypeStruct((H * NPAD, CP1), jnp.float32),
        compiler_params=pltpu.CompilerParams(needs_layout_passes=False,
                                             use_tc_tiling_on_sc=False),
        scratch_types=[
            pltpu.VMEM((NPAD,), jnp.float32),
            pltpu.VMEM((NPAD,), jnp.float32),
            pltpu.VMEM((NPAD,), jnp.float32),
            pltpu.VMEM((128,), jnp.int32),
            pltpu.VMEM((128,), jnp.int32),
            pltpu.VMEM((128,), jnp.int32),
            pltpu.VMEM((128,), jnp.float32),
            pltpu.VMEM((128, CP1), jnp.float32),
            pltpu.VMEM((128,), jnp.int32),
            pltpu.VMEM((128,), jnp.int32),
            pltpu.VMEM((128,), jnp.int32),
            pltpu.VMEM((128,), jnp.float32),
            pltpu.VMEM((128, CP1), jnp.float32),
            pltpu.VMEM_SHARED((NPAD, CP1), jnp.float32),
            pltpu.SemaphoreType.DMA,
            pltpu.SemaphoreType.DMA,
        ])(_sc1_body)
    return fn(srcp, dstp, asf, adf, bf, hpadf, acc0f)


def _tc2a_body(acc_ref, b1r_ref, w2_ref, as2r_ref, ad2r_ref,
               hpad2_ref, asad_ref):
    h2 = jnp.zeros((NBLK, C2), jnp.float32)
    w2 = w2_ref[...]
    for hd in range(H):
        acc = acc_ref[hd]
        hn = acc[:, 0:C1] / (acc[:, C1:C1 + 1] + 1e-30) + b1r_ref[hd, :][None, :]
        helu = jnp.where(hn > 0, hn, jnp.exp(hn) - 1.0)
        h2 = h2 + jnp.dot(helu, w2[hd * C1:(hd + 1) * C1, :],
                          preferred_element_type=jnp.float32, precision=lax.Precision.HIGHEST)
    as2 = jnp.sum(h2 * as2r_ref[0, :][None, :], axis=1, keepdims=True)
    ad2 = jnp.sum(h2 * ad2r_ref[0, :][None, :], axis=1, keepdims=True)
    hpad2_ref[...] = h2
    asad_ref[...] = jnp.concatenate([as2, ad2], axis=1)


def _tc2a(acc1, b1r, W2, a_src2, a_dst2):
    return pl.pallas_call(
        _tc2a_body,
        grid=(NPAD // NBLK,),
        in_specs=[
            pl.BlockSpec((H, NBLK, CP1), lambda i: (0, i, 0)),
            pl.BlockSpec((H, C1), lambda i: (0, 0)),
            pl.BlockSpec((H * C1, C2), lambda i: (0, 0)),
            pl.BlockSpec((1, C2), lambda i: (0, 0)),
            pl.BlockSpec((1, C2), lambda i: (0, 0)),
        ],
        out_specs=[
            pl.BlockSpec((NBLK, CP2), lambda i: (i, 0)),
            pl.BlockSpec((NBLK, 2), lambda i: (i, 0)),
        ],
        out_shape=[
            jax.ShapeDtypeStruct((NPAD, CP2), jnp.float32),
            jax.ShapeDtypeStruct((NPAD, 2), jnp.float32),
        ],
    )(acc1, b1r, W2, a_src2, a_dst2)


def _tc2b_body(asadf_ref, asadb_ref, hpad2_ref, b2c_ref, acc20_ref, den0_ref):
    A = jnp.max(asadf_ref[:, 0])
    asb = asadb_ref[:, 0:1]
    adb = asadb_ref[:, 1:2]
    B = _lrelu(adb + A)
    exs = jnp.exp(_lrelu(asb + adb) - B)
    acc20_ref[...] = hpad2_ref[...] * exs
    den0_ref[...] = exs
    b2c_ref[...] = B


def _tc2b(asad, hpad2):
    return pl.pallas_call(
        _tc2b_body,
        grid=(NPAD // NBLK,),
        in_specs=[
            pl.BlockSpec((NPAD, 2), lambda i: (0, 0)),
            pl.BlockSpec((NBLK, 2), lambda i: (i, 0)),
            pl.BlockSpec((NBLK, CP2), lambda i: (i, 0)),
        ],
        out_specs=[
            pl.BlockSpec((NBLK, 1), lambda i: (i, 0)),
            pl.BlockSpec((NBLK, CP2), lambda i: (i, 0)),
            pl.BlockSpec((NBLK, 1), lambda i: (i, 0)),
        ],
        out_shape=[
            jax.ShapeDtypeStruct((NPAD, 1), jnp.float32),
            jax.ShapeDtypeStruct((NPAD, CP2), jnp.float32),
            jax.ShapeDtypeStruct((NPAD, 1), jnp.float32),
        ],
    )(asad, asad, hpad2)


def _sc2_body(srcp, dstp, asf, adf, bf, hpadf, acc0f, den0f, out, dout,
              asv, adv, bv, srcv, dstv, exv, rows, sacc, sden, sem):
    c = lax.axis_index("c")
    s = lax.axis_index("s")
    g = c * 16 + s
    zero16 = jnp.zeros((16,), jnp.int32)
    pltpu.sync_copy(asf.at[pl.ds(0, NPAD)], asv)
    pltpu.sync_copy(adf.at[pl.ds(0, NPAD)], adv)
    pltpu.sync_copy(bf.at[pl.ds(0, NPAD)], bv)
    pltpu.sync_copy(acc0f.at[c, pl.ds(s * STRIPE, STRIPE)],
                    sacc.at[pl.ds(s * STRIPE, STRIPE)])
    pltpu.sync_copy(den0f.at[c, pl.ds(s * STRIPE, STRIPE)],
                    sden.at[pl.ds(s * STRIPE, STRIPE)])
    plsc.subcore_barrier()

    def chunk_body(ch, _):
        eb = g * EW2 + ch * 128
        pltpu.sync_copy(srcp.at[pl.ds(eb, 128)], srcv)
        pltpu.sync_copy(dstp.at[pl.ds(eb, 128)], dstv)
        for v in range(8):
            sl = pl.ds(v * 16, 16)
            si = srcv[sl]
            di = dstv[sl]
            a_s = plsc.load_gather(asv, [si])
            a_d = plsc.load_gather(adv, [di])
            b_d = plsc.load_gather(bv, [di])
            e = _lrelu(a_s + a_d)
            exv[sl] = jnp.exp(e - b_d)
        pltpu.async_copy(hpadf.at[srcv], rows, sem).wait()

        def scale_body(i2, _):
            spl = plsc.load_gather(exv, [zero16 + i2])
            for t in range(CP2 // 16):
                sl2 = pl.ds(t * 16, 16)
                rows[i2, sl2] = rows[i2, sl2] * spl
            return 0
        lax.fori_loop(0, 128, scale_body, 0)
        pltpu.sync_copy(rows, sacc.at[dstv], add=True)
        pltpu.sync_copy(exv, sden.at[dstv], add=True)
        return 0
    lax.fori_loop(0, NCH2, chunk_body, 0)
    plsc.subcore_barrier()
    pltpu.sync_copy(sacc.at[pl.ds(s * STRIPE, STRIPE)],
                    out.at[c, pl.ds(s * STRIPE, STRIPE)])
    pltpu.sync_copy(sden.at[pl.ds(s * STRIPE, STRIPE)],
                    dout.at[c, pl.ds(s * STRIPE, STRIPE)])


def _sc2(srcp, dstp, as2f, ad2f, b2f, hpad2, acc20full, den0full):
    mesh = plsc.VectorSubcoreMesh(core_axis_name="c", subcore_axis_name="s")
    fn = functools.partial(
        pl.kernel, mesh=mesh,
        out_type=[jax.ShapeDtypeStruct((2, NPAD, CP2), jnp.float32),
                  jax.ShapeDtypeStruct((2, NPAD), jnp.float32)],
        compiler_params=pltpu.CompilerParams(needs_layout_passes=False,
                                             use_tc_tiling_on_sc=False),
        scratch_types=[
            pltpu.VMEM((NPAD,), jnp.float32),
            pltpu.VMEM((NPAD,), jnp.float32),
            pltpu.VMEM((NPAD,), jnp.float32),
            pltpu.VMEM((128,), jnp.int32),
            pltpu.VMEM((128,), jnp.int32),
            pltpu.VMEM((128,), jnp.float32),
            pltpu.VMEM((128, CP2), jnp.float32),
            pltpu.VMEM_SHARED((NPAD, CP2), jnp.float32),
            pltpu.VMEM_SHARED((NPAD,), jnp.float32),
            pltpu.SemaphoreType.DMA,
        ])(_sc2_body)
    return fn(srcp, dstp, as2f, ad2f, b2f, hpad2, acc20full, den0full)


def _tc3a_body(a0_ref, a1_ref, d_ref, b2r_ref, ww_ref, bw_ref,
               h_ref, hw_ref):
    acc = a0_ref[0] + a1_ref[0]
    den = (d_ref[0, 0, :] + d_ref[0, 1, :])[:, None]
    h = acc / (den + 1e-30) + b2r_ref[0, :][None, :]
    h = jnp.maximum(h, 0.0)
    logit = jnp.dot(h, ww_ref[...], preferred_element_type=jnp.float32, precision=lax.Precision.HIGHEST) + bw_ref[0, 0]
    w = 1.0 / (1.0 + jnp.exp(-logit))
    h_ref[...] = h
    hw_ref[...] = h * w


def _tc3a(accL2, denL2, b2r, Ww, bw):
    return pl.pallas_call(
        _tc3a_body,
        grid=(NPAD // NBLK,),
        in_specs=[
            pl.BlockSpec((1, NBLK, CP2), lambda i: (0, i, 0)),
            pl.BlockSpec((1, NBLK, CP2), lambda i: (1, i, 0)),
            pl.BlockSpec((1, 2, NBLK), lambda i: (i, 0, 0)),
            pl.BlockSpec((1, C2), lambda i: (0, 0)),
            pl.BlockSpec((C2, 1), lambda i: (0, 0)),
            pl.BlockSpec((1, 1), lambda i: (0, 0)),
        ],
        out_specs=[
            pl.BlockSpec((NBLK, C2), lambda i: (i, 0)),
            pl.BlockSpec((NBLK, C2), lambda i: (i, 0)),
        ],
        out_shape=[
            jax.ShapeDtypeStruct((NPAD, C2), jnp.float32),
            jax.ShapeDtypeStruct((NPAD, C2), jnp.float32),
        ],
    )(accL2, accL2, denL2.reshape(2, NB, NBLK).transpose(1, 0, 2), b2r, Ww, bw)


def _tc3b_body(h_ref, hw_ref, batch_ref, batchc_ref, hsum_ref, hmax8_ref):
    bt = batch_ref[...]
    gi = lax.broadcasted_iota(jnp.int32, (G, NPAD), 0)
    oh = (bt == gi)
    hsum_ref[...] = jnp.dot(oh.astype(jnp.float32), hw_ref[...],
                            preferred_element_type=jnp.float32, precision=lax.Precision.HIGHEST)
    h = h_ref[...]
    btc = batchc_ref[...]
    ninf = jnp.float32(-jnp.inf)

    def body(g, _):
        mask = (btc == g)
        row = jnp.max(jnp.where(mask, h, ninf), axis=0, keepdims=True)
        row = jnp.where(jnp.isfinite(row), row, 0.0)
        hmax8_ref[pl.ds(pl.multiple_of(g * 8, 8), 8), :] = (
            jnp.broadcast_to(row, (8, C2)))
        return 0
    lax.fori_loop(0, G, body, 0)


def _tc3b(h, hw, batchr):
    return pl.pallas_call(
        _tc3b_body,
        grid=(1,),
        in_specs=[
            pl.BlockSpec((NPAD, C2), lambda i: (0, 0)),
            pl.BlockSpec((NPAD, C2), lambda i: (0, 0)),
            pl.BlockSpec((1, NPAD), lambda i: (0, 0)),
            pl.BlockSpec((NPAD, 1), lambda i: (0, 0)),
        ],
        out_specs=[
            pl.BlockSpec((G, C2), lambda i: (0, 0)),
            pl.BlockSpec((G * 8, C2), lambda i: (0, 0)),
        ],
        out_shape=[
            jax.ShapeDtypeStruct((G, C2), jnp.float32),
            jax.ShapeDtypeStruct((G * 8, C2), jnp.float32),
        ],
    )(h, hw, batchr, batchr.reshape(NPAD, 1))


def _tc3c_body(hg_ref, te_ref, wxt_ref, bxt_ref, w1_ref, bf1_ref,
               w2_ref, bf2_ref, wo_ref, bo_ref, out_ref):
    xt = jnp.dot(te_ref[...], wxt_ref[...],
                 preferred_element_type=jnp.float32, precision=lax.Precision.HIGHEST) + bxt_ref[0, :][None, :]
    xc = jnp.concatenate([hg_ref[...], xt], axis=1)
    y = jnp.maximum(jnp.dot(xc, w1_ref[...], preferred_element_type=jnp.float32, precision=lax.Precision.HIGHEST)
                    + bf1_ref[0, :][None, :], 0.0)
    y = jnp.maximum(jnp.dot(y, w2_ref[...], preferred_element_type=jnp.float32, precision=lax.Precision.HIGHEST)
                    + bf2_ref[0, :][None, :], 0.0)
    out_ref[...] = jnp.dot(y, wo_ref[...],
                           preferred_element_type=jnp.float32, precision=lax.Precision.HIGHEST) + bo_ref[0, 0]


def _tc3c(hg, te, Wxt, bxt, Wfc1, bfc1, Wfc2, bfc2, Wout, bout):
    specs = [
        pl.BlockSpec((G, 2 * C2), lambda i: (0, 0)),
        pl.BlockSpec((G, 1280), lambda i: (0, 0)),
        pl.BlockSpec((1280, 256), lambda i: (0, 0)),
        pl.BlockSpec((1, 256), lambda i: (0, 0)),
        pl.BlockSpec((512, 1024), lambda i: (0, 0)),
        pl.BlockSpec((1, 1024), lambda i: (0, 0)),
        pl.BlockSpec((1024, 256), lambda i: (0, 0)),
        pl.BlockSpec((1, 256), lambda i: (0, 0)),
        pl.BlockSpec((256, 1), lambda i: (0, 0)),
        pl.BlockSpec((1, 1), lambda i: (0, 0)),
    ]
    return pl.pallas_call(
        _tc3c_body,
        grid=(1,),
        in_specs=specs,
        out_specs=pl.BlockSpec((G, 1), lambda i: (0, 0)),
        out_shape=jax.ShapeDtypeStruct((G, 1), jnp.float32),
    )(hg, te, Wxt, bxt.reshape(1, 256), Wfc1, bfc1.reshape(1, 1024),
      Wfc2, bfc2.reshape(1, 256), Wout, bout.reshape(1, 1))


def kernel(x, edge_index, batch, target_embedding, W1, a_src1, a_dst1, b1,
           W2, a_src2, a_dst2, b2, Ww, bw, Wxt, bxt, Wfc1, bfc1, Wfc2, bfc2,
           Wout, bout):
    xpad = jnp.zeros((NPAD, C1), jnp.float32).at[:N].set(x)
    ei = edge_index.astype(jnp.int32)
    srcp = jnp.full((EPAD,), SENT, jnp.int32).at[:E].set(ei[0])
    dstp = jnp.full((EPAD,), SENT, jnp.int32).at[:E].set(ei[1])

    hpad, as1, ad1 = _tc1a(xpad, W1, a_src1, a_dst1)
    B1, acc0 = _tc1b(as1, ad1, hpad)
    asf = as1.transpose(1, 0, 2).reshape(-1)
    adf = ad1.transpose(1, 0, 2).reshape(-1)
    bf = B1.transpose(1, 0, 2).reshape(-1)
    acc1 = _sc1(srcp, dstp, asf, adf, bf,
                hpad.reshape(H * NPAD, CP1), acc0.reshape(H * NPAD, CP1))
    acc1 = acc1.reshape(H, NPAD, CP1)

    hpad2, asad2 = _tc2a(acc1, b1.reshape(H, C1), W2, a_src2, a_dst2)
    B2, acc20, den0 = _tc2b(asad2, hpad2)
    acc20full = jnp.stack([acc20, jnp.zeros_like(acc20)], axis=0)
    den0full = jnp.stack([den0.reshape(-1), jnp.zeros((NPAD,), jnp.float32)],
                         axis=0)
    accL2, denL2 = _sc2(srcp, dstp, asad2[:, 0], asad2[:, 1], B2.reshape(-1),
                        hpad2, acc20full, den0full)

    batchr = jnp.full((1, NPAD), jnp.int32(1 << 30)).at[0, :N].set(
        batch.astype(jnp.int32))
    h, hw = _tc3a(accL2, denL2, b2.reshape(1, C2), Ww, bw.reshape(1, 1))
    hsum, hmax8 = _tc3b(h, hw, batchr)
    hg = jnp.concatenate([hsum, hmax8.reshape(G, 8, C2)[:, 0, :]], axis=1)
    return _tc3c(hg, target_embedding, Wxt, bxt, Wfc1, bfc1, Wfc2, bfc2,
                 Wout, bout)


# speedup vs baseline: 15.3870x; 1.0782x over previous
import functools
import jax
import jax.numpy as jnp
from jax import lax
from jax.experimental import pallas as pl
from jax.experimental.pallas import tpu as pltpu
from jax.experimental.pallas import tpu_sc as plsc

N = 10000
NPAD = 10112
E = 160000
H = 10
C1 = 78
CP1 = 80
C2 = 128
CP2 = 128
G = 64
NBLK = 1264
NB = 8
EWP1 = 10112
NCH1 = 79
EW2 = 5120
NCH2 = 40
EPAD = 163840
SENT = 10000
STRIPE = NPAD // 16


def _lrelu(x):
    return jnp.where(x > 0, x, 0.2 * x)


def _tc1a_body(x_ref, w1_ref, asr_ref, adr_ref, hpad_ref, as1_ref, ad1_ref):
    x = x_ref[...]
    w1 = w1_ref[...]
    ones = jnp.ones((NBLK, 1), jnp.float32)
    zeros = jnp.zeros((NBLK, 1), jnp.float32)
    as_rows = []
    ad_rows = []
    for hd in range(H):
        w = w1[:, hd * C1:(hd + 1) * C1]
        hh = jnp.dot(x, w, preferred_element_type=jnp.float32, precision=lax.Precision.HIGHEST)
        hpad_ref[hd] = jnp.concatenate([hh, ones, zeros], axis=1)
        as_rows.append(jnp.sum(hh * asr_ref[hd, :][None, :], axis=1))
        ad_rows.append(jnp.sum(hh * adr_ref[hd, :][None, :], axis=1))
    as1_ref[0] = jnp.stack(as_rows, axis=0)
    ad1_ref[0] = jnp.stack(ad_rows, axis=0)


def _tc1a(xpad, W1, a_src1, a_dst1):
    return pl.pallas_call(
        _tc1a_body,
        grid=(NPAD // NBLK,),
        in_specs=[
            pl.BlockSpec((NBLK, C1), lambda i: (i, 0)),
            pl.BlockSpec((C1, H * C1), lambda i: (0, 0)),
            pl.BlockSpec((H, C1), lambda i: (0, 0)),
            pl.BlockSpec((H, C1), lambda i: (0, 0)),
        ],
        out_specs=[
            pl.BlockSpec((H, NBLK, CP1), lambda i: (0, i, 0)),
            pl.BlockSpec((1, H, NBLK), lambda i: (i, 0, 0)),
            pl.BlockSpec((1, H, NBLK), lambda i: (i, 0, 0)),
        ],
        out_shape=[
            jax.ShapeDtypeStruct((H, NPAD, CP1), jnp.float32),
            jax.ShapeDtypeStruct((NB, H, NBLK), jnp.float32),
            jax.ShapeDtypeStruct((NB, H, NBLK), jnp.float32),
        ],
    )(xpad, W1, a_src1, a_dst1)


def _tc1b_body(asf_ref, asb_ref, adb_ref, hpad_ref, b1_ref, acc0_ref):
    A = jnp.max(asf_ref[...], axis=(0, 2))
    asb = asb_ref[0]
    adb = adb_ref[0]
    B = _lrelu(adb + A[:, None])
    exs = jnp.exp(_lrelu(asb + adb) - B)
    for hd in range(H):
        acc0_ref[hd] = hpad_ref[hd] * exs[hd, :][:, None]
    b1_ref[0] = B


def _tc1b(as1, ad1, hpad):
    return pl.pallas_call(
        _tc1b_body,
        grid=(NPAD // NBLK,),
        in_specs=[
            pl.BlockSpec((NB, H, NBLK), lambda i: (0, 0, 0)),
            pl.BlockSpec((1, H, NBLK), lambda i: (i, 0, 0)),
            pl.BlockSpec((1, H, NBLK), lambda i: (i, 0, 0)),
            pl.BlockSpec((H, NBLK, CP1), lambda i: (0, i, 0)),
        ],
        out_specs=[
            pl.BlockSpec((1, H, NBLK), lambda i: (i, 0, 0)),
            pl.BlockSpec((H, NBLK, CP1), lambda i: (0, i, 0)),
        ],
        out_shape=[
            jax.ShapeDtypeStruct((NB, H, NBLK), jnp.float32),
            jax.ShapeDtypeStruct((H, NPAD, CP1), jnp.float32),
        ],
    )(as1, as1, ad1, hpad)


def _sc1_body(srcp, dstp, asf, adf, bf, hpadf, acc0f, out,
              asv, adv, bv,
              srcv0, dstv0, gidx0, exv0, rows0,
              srcv1, dstv1, gidx1, exv1, rows1,
              srcv2, dstv2, gidx2, exv2, rows2,
              srcv3, dstv3, gidx3, exv3, rows3,
              sacc, semg0, semg1, semg2, semg3, sems0, sems1, sems2, sems3):
    c = lax.axis_index("c")
    s = lax.axis_index("s")
    zero16 = jnp.zeros((16,), jnp.int32)
    bufs = ((srcv0, dstv0, gidx0, exv0, rows0, semg0, sems0),
            (srcv1, dstv1, gidx1, exv1, rows1, semg1, sems1),
            (srcv2, dstv2, gidx2, exv2, rows2, semg2, sems2),
            (srcv3, dstv3, gidx3, exv3, rows3, semg3, sems3))
    for hi in range(H // 2):
        hd = c * (H // 2) + hi
        base = hd * NPAD
        pltpu.sync_copy(asf.at[pl.ds(base, NPAD)], asv)
        pltpu.sync_copy(adf.at[pl.ds(base, NPAD)], adv)
        pltpu.sync_copy(bf.at[pl.ds(base, NPAD)], bv)
        pltpu.sync_copy(acc0f.at[pl.ds(base + s * STRIPE, STRIPE)],
                        sacc.at[pl.ds(s * STRIPE, STRIPE)])
        plsc.subcore_barrier()

        def stage(ch, b, first_round, base=base):
            srcv, dstv, gidx, exv, rows, semg, sems = bufs[b]

            @pl.when(ch < NCH1)
            def _():
                if not first_round:
                    pltpu.make_async_copy(rows, sacc.at[dstv], sems).wait()
                eb = s * EWP1 + ch * 128
                pltpu.sync_copy(srcp.at[pl.ds(eb, 128)], srcv)
                pltpu.sync_copy(dstp.at[pl.ds(eb, 128)], dstv)
                for v in range(8):
                    sl = pl.ds(v * 16, 16)
                    si = srcv[sl]
                    di = dstv[sl]
                    a_s = plsc.load_gather(asv, [si])
                    a_d = plsc.load_gather(adv, [di])
                    b_d = plsc.load_gather(bv, [di])
                    e = _lrelu(a_s + a_d)
                    exv[sl] = jnp.exp(e - b_d)
                    gidx[sl] = si + base
                pltpu.async_copy(hpadf.at[gidx], rows, semg)

        def process(ch, b):
            srcv, dstv, gidx, exv, rows, semg, sems = bufs[b]

            @pl.when(ch < NCH1)
            def _():
                pltpu.make_async_copy(hpadf.at[gidx], rows, semg).wait()

                def scale_body(i2, _):
                    spl = plsc.load_gather(exv, [zero16 + i2])
                    for t in range(CP1 // 16):
                        sl2 = pl.ds(t * 16, 16)
                        rows[i2, sl2] = rows[i2, sl2] * spl
                    return 0
                lax.fori_loop(0, 128, scale_body, 0)
                pltpu.async_copy(rows, sacc.at[dstv], sems, add=True)

        stage(0, 0, True)
        stage(1, 1, True)
        stage(2, 2, True)
        stage(3, 3, True)

        def super_body(ss, _):
            ch = ss * 4
            process(ch, 0)
            process(ch + 1, 1)
            stage(ch + 4, 0, False)
            process(ch + 2, 2)
            stage(ch + 5, 1, False)
            process(ch + 3, 3)
            stage(ch + 6, 2, False)
            stage(ch + 7, 3, False)
            return 0
        lax.fori_loop(0, (NCH1 + 3) // 4, super_body, 0)
        for b in (0, 1, 2, 3):
            srcv, dstv, gidx, exv, rows, semg, sems = bufs[b]
            pltpu.make_async_copy(rows, sacc.at[dstv], sems).wait()
        plsc.subcore_barrier()
        pltpu.sync_copy(sacc.at[pl.ds(s * STRIPE, STRIPE)],
                        out.at[pl.ds(base + s * STRIPE, STRIPE)])
        plsc.subcore_barrier()


def _sc1(srcp, dstp, asf, adf, bf, hpadf, acc0f):
    mesh = plsc.VectorSubcoreMesh(core_axis_name="c", subcore_axis_name="s")
    fn = functools.partial(
        pl.kernel, mesh=mesh,
        out_type=jax.ShapeDtypeStruct((H * NPAD, CP1), jnp.float32),
        compiler_params=pltpu.CompilerParams(needs_layout_passes=False,
                                             use_tc_tiling_on_sc=False),
        scratch_types=[
            pltpu.VMEM((NPAD,), jnp.float32),
            pltpu.VMEM((NPAD,), jnp.float32),
            pltpu.VMEM((NPAD,), jnp.float32),
            pltpu.VMEM((128,), jnp.int32),
            pltpu.VMEM((128,), jnp.int32),
            pltpu.VMEM((128,), jnp.int32),
            pltpu.VMEM((128,), jnp.float32),
            pltpu.VMEM((128, CP1), jnp.float32),
            pltpu.VMEM((128,), jnp.int32),
            pltpu.VMEM((128,), jnp.int32),
            pltpu.VMEM((128,), jnp.int32),
            pltpu.VMEM((128,), jnp.float32),
            pltpu.VMEM((128, CP1), jnp.float32),
            pltpu.VMEM((128,), jnp.int32),
            pltpu.VMEM((128,), jnp.int32),
            pltpu.VMEM((128,), jnp.int32),
            pltpu.VMEM((128,), jnp.float32),
            pltpu.VMEM((128, CP1), jnp.float32),
            pltpu.VMEM((128,), jnp.int32),
            pltpu.VMEM((128,), jnp.int32),
            pltpu.VMEM((128,), jnp.int32),
            pltpu.VMEM((128,), jnp.float32),
            pltpu.VMEM((128, CP1), jnp.float32),
            pltpu.VMEM_SHARED((NPAD, CP1), jnp.float32),
            pltpu.SemaphoreType.DMA,
            pltpu.SemaphoreType.DMA,
            pltpu.SemaphoreType.DMA,
            pltpu.SemaphoreType.DMA,
            pltpu.SemaphoreType.DMA,
            pltpu.SemaphoreType.DMA,
            pltpu.SemaphoreType.DMA,
            pltpu.SemaphoreType.DMA,
        ])(_sc1_body)
    return fn(srcp, dstp, asf, adf, bf, hpadf, acc0f)


def _tc2a_body(acc_ref, b1r_ref, w2_ref, as2r_ref, ad2r_ref,
               hpad2_ref, asad_ref):
    h2 = jnp.zeros((NBLK, C2), jnp.float32)
    w2 = w2_ref[...]
    for hd in range(H):
        acc = acc_ref[hd]
        hn = acc[:, 0:C1] / (acc[:, C1:C1 + 1] + 1e-30) + b1r_ref[hd, :][None, :]
        helu = jnp.where(hn > 0, hn, jnp.exp(hn) - 1.0)
        h2 = h2 + jnp.dot(helu, w2[hd * C1:(hd + 1) * C1, :],
                          preferred_element_type=jnp.float32, precision=lax.Precision.HIGHEST)
    as2 = jnp.sum(h2 * as2r_ref[0, :][None, :], axis=1, keepdims=True)
    ad2 = jnp.sum(h2 * ad2r_ref[0, :][None, :], axis=1, keepdims=True)
    hpad2_ref[...] = h2
    asad_ref[...] = jnp.concatenate([as2, ad2], axis=1)


def _tc2a(acc1, b1r, W2, a_src2, a_dst2):
    return pl.pallas_call(
        _tc2a_body,
        grid=(NPAD // NBLK,),
        in_specs=[
            pl.BlockSpec((H, NBLK, CP1), lambda i: (0, i, 0)),
            pl.BlockSpec((H, C1), lambda i: (0, 0)),
            pl.BlockSpec((H * C1, C2), lambda i: (0, 0)),
            pl.BlockSpec((1, C2), lambda i: (0, 0)),
            pl.BlockSpec((1, C2), lambda i: (0, 0)),
        ],
        out_specs=[
            pl.BlockSpec((NBLK, CP2), lambda i: (i, 0)),
            pl.BlockSpec((NBLK, 2), lambda i: (i, 0)),
        ],
        out_shape=[
            jax.ShapeDtypeStruct((NPAD, CP2), jnp.float32),
            jax.ShapeDtypeStruct((NPAD, 2), jnp.float32),
        ],
    )(acc1, b1r, W2, a_src2, a_dst2)


def _tc2b_body(asadf_ref, asadb_ref, hpad2_ref, b2c_ref, acc20_ref, den0_ref):
    A = jnp.max(asadf_ref[:, 0])
    asb = asadb_ref[:, 0:1]
    adb = asadb_ref[:, 1:2]
    B = _lrelu(adb + A)
    exs = jnp.exp(_lrelu(asb + adb) - B)
    acc20_ref[...] = hpad2_ref[...] * exs
    den0_ref[...] = exs
    b2c_ref[...] = B


def _tc2b(asad, hpad2):
    return pl.pallas_call(
        _tc2b_body,
        grid=(NPAD // NBLK,),
        in_specs=[
            pl.BlockSpec((NPAD, 2), lambda i: (0, 0)),
            pl.BlockSpec((NBLK, 2), lambda i: (i, 0)),
            pl.BlockSpec((NBLK, CP2), lambda i: (i, 0)),
        ],
        out_specs=[
            pl.BlockSpec((NBLK, 1), lambda i: (i, 0)),
            pl.BlockSpec((NBLK, CP2), lambda i: (i, 0)),
            pl.BlockSpec((NBLK, 1), lambda i: (i, 0)),
        ],
        out_shape=[
            jax.ShapeDtypeStruct((NPAD, 1), jnp.float32),
            jax.ShapeDtypeStruct((NPAD, CP2), jnp.float32),
            jax.ShapeDtypeStruct((NPAD, 1), jnp.float32),
        ],
    )(asad, asad, hpad2)


def _sc2_body(srcp, dstp, asf, adf, bf, hpadf, acc0f, den0f, out, dout,
              asv, adv, bv, srcv, dstv, exv, rows, sacc, sden, sem):
    c = lax.axis_index("c")
    s = lax.axis_index("s")
    g = c * 16 + s
    zero16 = jnp.zeros((16,), jnp.int32)
    pltpu.sync_copy(asf.at[pl.ds(0, NPAD)], asv)
    pltpu.sync_copy(adf.at[pl.ds(0, NPAD)], adv)
    pltpu.sync_copy(bf.at[pl.ds(0, NPAD)], bv)
    pltpu.sync_copy(acc0f.at[c, pl.ds(s * STRIPE, STRIPE)],
                    sacc.at[pl.ds(s * STRIPE, STRIPE)])
    pltpu.sync_copy(den0f.at[c, pl.ds(s * STRIPE, STRIPE)],
                    sden.at[pl.ds(s * STRIPE, STRIPE)])
    plsc.subcore_barrier()

    def chunk_body(ch, _):
        eb = g * EW2 + ch * 128
        pltpu.sync_copy(srcp.at[pl.ds(eb, 128)], srcv)
        pltpu.sync_copy(dstp.at[pl.ds(eb, 128)], dstv)
        for v in range(8):
            sl = pl.ds(v * 16, 16)
            si = srcv[sl]
            di = dstv[sl]
            a_s = plsc.load_gather(asv, [si])
            a_d = plsc.load_gather(adv, [di])
            b_d = plsc.load_gather(bv, [di])
            e = _lrelu(a_s + a_d)
            exv[sl] = jnp.exp(e - b_d)
        pltpu.async_copy(hpadf.at[srcv], rows, sem).wait()

        def scale_body(i2, _):
            spl = plsc.load_gather(exv, [zero16 + i2])
            for t in range(CP2 // 16):
                sl2 = pl.ds(t * 16, 16)
                rows[i2, sl2] = rows[i2, sl2] * spl
            return 0
        lax.fori_loop(0, 128, scale_body, 0)
        pltpu.sync_copy(rows, sacc.at[dstv], add=True)
        pltpu.sync_copy(exv, sden.at[dstv], add=True)
        return 0
    lax.fori_loop(0, NCH2, chunk_body, 0)
    plsc.subcore_barrier()
    pltpu.sync_copy(sacc.at[pl.ds(s * STRIPE, STRIPE)],
                    out.at[c, pl.ds(s * STRIPE, STRIPE)])
    pltpu.sync_copy(sden.at[pl.ds(s * STRIPE, STRIPE)],
                    dout.at[c, pl.ds(s * STRIPE, STRIPE)])


def _sc2(srcp, dstp, as2f, ad2f, b2f, hpad2, acc20full, den0full):
    mesh = plsc.VectorSubcoreMesh(core_axis_name="c", subcore_axis_name="s")
    fn = functools.partial(
        pl.kernel, mesh=mesh,
        out_type=[jax.ShapeDtypeStruct((2, NPAD, CP2), jnp.float32),
                  jax.ShapeDtypeStruct((2, NPAD), jnp.float32)],
        compiler_params=pltpu.CompilerParams(needs_layout_passes=False,
                                             use_tc_tiling_on_sc=False),
        scratch_types=[
            pltpu.VMEM((NPAD,), jnp.float32),
            pltpu.VMEM((NPAD,), jnp.float32),
            pltpu.VMEM((NPAD,), jnp.float32),
            pltpu.VMEM((128,), jnp.int32),
            pltpu.VMEM((128,), jnp.int32),
            pltpu.VMEM((128,), jnp.float32),
            pltpu.VMEM((128, CP2), jnp.float32),
            pltpu.VMEM_SHARED((NPAD, CP2), jnp.float32),
            pltpu.VMEM_SHARED((NPAD,), jnp.float32),
            pltpu.SemaphoreType.DMA,
        ])(_sc2_body)
    return fn(srcp, dstp, as2f, ad2f, b2f, hpad2, acc20full, den0full)


def _tc3a_body(a0_ref, a1_ref, d_ref, b2r_ref, ww_ref, bw_ref,
               h_ref, hw_ref):
    acc = a0_ref[0] + a1_ref[0]
    den = (d_ref[0, 0, :] + d_ref[0, 1, :])[:, None]
    h = acc / (den + 1e-30) + b2r_ref[0, :][None, :]
    h = jnp.maximum(h, 0.0)
    logit = jnp.dot(h, ww_ref[...], preferred_element_type=jnp.float32, precision=lax.Precision.HIGHEST) + bw_ref[0, 0]
    w = 1.0 / (1.0 + jnp.exp(-logit))
    h_ref[...] = h
    hw_ref[...] = h * w


def _tc3a(accL2, denL2, b2r, Ww, bw):
    return pl.pallas_call(
        _tc3a_body,
        grid=(NPAD // NBLK,),
        in_specs=[
            pl.BlockSpec((1, NBLK, CP2), lambda i: (0, i, 0)),
            pl.BlockSpec((1, NBLK, CP2), lambda i: (1, i, 0)),
            pl.BlockSpec((1, 2, NBLK), lambda i: (i, 0, 0)),
            pl.BlockSpec((1, C2), lambda i: (0, 0)),
            pl.BlockSpec((C2, 1), lambda i: (0, 0)),
            pl.BlockSpec((1, 1), lambda i: (0, 0)),
        ],
        out_specs=[
            pl.BlockSpec((NBLK, C2), lambda i: (i, 0)),
            pl.BlockSpec((NBLK, C2), lambda i: (i, 0)),
        ],
        out_shape=[
            jax.ShapeDtypeStruct((NPAD, C2), jnp.float32),
            jax.ShapeDtypeStruct((NPAD, C2), jnp.float32),
        ],
    )(accL2, accL2, denL2.reshape(2, NB, NBLK).transpose(1, 0, 2), b2r, Ww, bw)


def _tc3b_body(h_ref, hw_ref, batch_ref, batchc_ref, hsum_ref, hmax8_ref):
    bt = batch_ref[...]
    gi = lax.broadcasted_iota(jnp.int32, (G, NPAD), 0)
    oh = (bt == gi)
    hsum_ref[...] = jnp.dot(oh.astype(jnp.float32), hw_ref[...],
                            preferred_element_type=jnp.float32, precision=lax.Precision.HIGHEST)
    h = h_ref[...]
    btc = batchc_ref[...]
    ninf = jnp.float32(-jnp.inf)

    def body(g, _):
        mask = (btc == g)
        row = jnp.max(jnp.where(mask, h, ninf), axis=0, keepdims=True)
        row = jnp.where(jnp.isfinite(row), row, 0.0)
        hmax8_ref[pl.ds(pl.multiple_of(g * 8, 8), 8), :] = (
            jnp.broadcast_to(row, (8, C2)))
        return 0
    lax.fori_loop(0, G, body, 0)


def _tc3b(h, hw, batchr):
    return pl.pallas_call(
        _tc3b_body,
        grid=(1,),
        in_specs=[
            pl.BlockSpec((NPAD, C2), lambda i: (0, 0)),
            pl.BlockSpec((NPAD, C2), lambda i: (0, 0)),
            pl.BlockSpec((1, NPAD), lambda i: (0, 0)),
            pl.BlockSpec((NPAD, 1), lambda i: (0, 0)),
        ],
        out_specs=[
            pl.BlockSpec((G, C2), lambda i: (0, 0)),
            pl.BlockSpec((G * 8, C2), lambda i: (0, 0)),
        ],
        out_shape=[
            jax.ShapeDtypeStruct((G, C2), jnp.float32),
            jax.ShapeDtypeStruct((G * 8, C2), jnp.float32),
        ],
    )(h, hw, batchr, batchr.reshape(NPAD, 1))


def _tc3c_body(hg_ref, te_ref, wxt_ref, bxt_ref, w1_ref, bf1_ref,
               w2_ref, bf2_ref, wo_ref, bo_ref, out_ref):
    xt = jnp.dot(te_ref[...], wxt_ref[...],
                 preferred_element_type=jnp.float32, precision=lax.Precision.HIGHEST) + bxt_ref[0, :][None, :]
    xc = jnp.concatenate([hg_ref[...], xt], axis=1)
    y = jnp.maximum(jnp.dot(xc, w1_ref[...], preferred_element_type=jnp.float32, precision=lax.Precision.HIGHEST)
                    + bf1_ref[0, :][None, :], 0.0)
    y = jnp.maximum(jnp.dot(y, w2_ref[...], preferred_element_type=jnp.float32, precision=lax.Precision.HIGHEST)
                    + bf2_ref[0, :][None, :], 0.0)
    out_ref[...] = jnp.dot(y, wo_ref[...],
                           preferred_element_type=jnp.float32, precision=lax.Precision.HIGHEST) + bo_ref[0, 0]


def _tc3c(hg, te, Wxt, bxt, Wfc1, bfc1, Wfc2, bfc2, Wout, bout):
    specs = [
        pl.BlockSpec((G, 2 * C2), lambda i: (0, 0)),
        pl.BlockSpec((G, 1280), lambda i: (0, 0)),
        pl.BlockSpec((1280, 256), lambda i: (0, 0)),
        pl.BlockSpec((1, 256), lambda i: (0, 0)),
        pl.BlockSpec((512, 1024), lambda i: (0, 0)),
        pl.BlockSpec((1, 1024), lambda i: (0, 0)),
        pl.BlockSpec((1024, 256), lambda i: (0, 0)),
        pl.BlockSpec((1, 256), lambda i: (0, 0)),
        pl.BlockSpec((256, 1), lambda i: (0, 0)),
        pl.BlockSpec((1, 1), lambda i: (0, 0)),
    ]
    return pl.pallas_call(
        _tc3c_body,
        grid=(1,),
        in_specs=specs,
        out_specs=pl.BlockSpec((G, 1), lambda i: (0, 0)),
        out_shape=jax.ShapeDtypeStruct((G, 1), jnp.float32),
    )(hg, te, Wxt, bxt.reshape(1, 256), Wfc1, bfc1.reshape(1, 1024),
      Wfc2, bfc2.reshape(1, 256), Wout, bout.reshape(1, 1))


def kernel(x, edge_index, batch, target_embedding, W1, a_src1, a_dst1, b1,
           W2, a_src2, a_dst2, b2, Ww, bw, Wxt, bxt, Wfc1, bfc1, Wfc2, bfc2,
           Wout, bout):
    xpad = jnp.zeros((NPAD, C1), jnp.float32).at[:N].set(x)
    ei = edge_index.astype(jnp.int32)
    srcp = jnp.full((EPAD,), SENT, jnp.int32).at[:E].set(ei[0])
    dstp = jnp.full((EPAD,), SENT, jnp.int32).at[:E].set(ei[1])

    hpad, as1, ad1 = _tc1a(xpad, W1, a_src1, a_dst1)
    B1, acc0 = _tc1b(as1, ad1, hpad)
    asf = as1.transpose(1, 0, 2).reshape(-1)
    adf = ad1.transpose(1, 0, 2).reshape(-1)
    bf = B1.transpose(1, 0, 2).reshape(-1)
    acc1 = _sc1(srcp, dstp, asf, adf, bf,
                hpad.reshape(H * NPAD, CP1), acc0.reshape(H * NPAD, CP1))
    acc1 = acc1.reshape(H, NPAD, CP1)

    hpad2, asad2 = _tc2a(acc1, b1.reshape(H, C1), W2, a_src2, a_dst2)
    B2, acc20, den0 = _tc2b(asad2, hpad2)
    acc20full = jnp.stack([acc20, jnp.zeros_like(acc20)], axis=0)
    den0full = jnp.stack([den0.reshape(-1), jnp.zeros((NPAD,), jnp.float32)],
                         axis=0)
    accL2, denL2 = _sc2(srcp, dstp, asad2[:, 0], asad2[:, 1], B2.reshape(-1),
                        hpad2, acc20full, den0full)

    batchr = jnp.full((1, NPAD), jnp.int32(1 << 30)).at[0, :N].set(
        batch.astype(jnp.int32))
    h, hw = _tc3a(accL2, denL2, b2.reshape(1, C2), Ww, bw.reshape(1, 1))
    hsum, hmax8 = _tc3b(h, hw, batchr)
    hg = jnp.concatenate([hsum, hmax8.reshape(G, 8, C2)[:, 0, :]], axis=1)
    return _tc3c(hg, target_embedding, Wxt, bxt, Wfc1, bfc1, Wfc2, bfc2,
                 Wout, bout)
